# baseline (speedup 1.0000x reference)
import functools

import jax
import jax.numpy as jnp
from jax import lax
from jax.experimental import pallas as pl
from jax.experimental.pallas import tpu as pltpu
from jax.experimental.pallas import tpu_sc as plsc

_NC = 2
_NS = 16
_CH = 80


def _deg_body(npad, depw, dsteps, dst_hbm, out_hbm, didx, ones_v, zb, deg_sh,
              isem0, isem1, isem2, isem3):
    c = lax.axis_index("c")
    s = lax.axis_index("s")
    isems = (isem0, isem1, isem2, isem3)
    one16 = jnp.ones((16,), jnp.float32)
    z16 = jnp.zeros((16,), jnp.float32)
    for i in range(_CH):
        for j in range(8):
            ones_v[i, pl.ds(16 * j, 16)] = one16
    for i in range(32):
        for j in range(8):
            zb[i, pl.ds(16 * j, 16)] = z16
    rps = npad // _NS

    def zstep(k, carry):
        pltpu.sync_copy(zb, deg_sh.at[pl.ds(s * rps + k * 32, 32)])
        return carry

    lax.fori_loop(0, rps // 32, zstep, 0)
    plsc.subcore_barrier()
    wid = s * _NC + c
    ebase = wid * depw

    def issue_idx(i, slot):
        pltpu.async_copy(dst_hbm.at[pl.ds(ebase + i * _CH, _CH)],
                         didx.at[slot], isems[slot])

    def wait_idx(slot):
        pltpu.make_async_copy(dst_hbm.at[pl.ds(0, _CH)], didx.at[slot],
                              isems[slot]).wait()

    def scatter(slot):
        pltpu.sync_copy(ones_v, deg_sh.at[didx.at[slot]], add=True)

    issue_idx(0, 0)
    issue_idx(1, 1)

    def quad(j, carry):
        for u in range(4):
            issue_idx(4 * j + u + 2, (u + 2) % 4)
            wait_idx(u)
            scatter(u)
        return carry

    nmain = 4 * ((dsteps - 2) // 4)
    lax.fori_loop(0, nmain // 4, quad, 0)
    for t in range(nmain, dsteps):
        if t + 2 < dsteps:
            issue_idx(t + 2, (t + 2) % 4)
        wait_idx(t % 4)
        scatter(t % 4)

    plsc.subcore_barrier()
    pltpu.sync_copy(deg_sh.at[pl.ds(s * rps, rps)],
                    out_hbm.at[pl.ds(c * npad + s * rps, rps)])


def _make_deg(npad, ee):
    depw = ee // (_NC * _NS)
    dsteps = depw // _CH
    mesh = plsc.VectorSubcoreMesh(core_axis_name="c", subcore_axis_name="s")
    return functools.partial(
        pl.kernel,
        functools.partial(_deg_body, npad, depw, dsteps),
        mesh=mesh,
        out_type=[jax.ShapeDtypeStruct((2 * npad, 128), jnp.float32)],
        scratch_types=[
            pltpu.VMEM((4, _CH), jnp.int32),
            pltpu.VMEM((_CH, 128), jnp.float32),
            pltpu.VMEM((32, 128), jnp.float32),
            pltpu.VMEM_SHARED((npad, 128), jnp.float32),
            pltpu.SemaphoreType.DMA,
            pltpu.SemaphoreType.DMA,
            pltpu.SemaphoreType.DMA,
            pltpu.SemaphoreType.DMA,
        ],
    )()


def _conv_body(npad, eps, steps, h_hbm, src_hbm, dst_hbm, out_hbm,
               sidxa, sidxb, didxa, didxb, rowsa, rowsb, zbuf, acc_sh,
               isem0, isem1, isem2, isem3, gsem0, gsem1, ssem0, ssem1):
    c = lax.axis_index("c")
    s = lax.axis_index("s")
    isems = (isem0, isem1, isem2, isem3)
    gsems = (gsem0, gsem1)
    ssems = (ssem0, ssem1)
    z16 = jnp.zeros((16,), jnp.float32)
    for i in range(32):
        for j in range(8):
            zbuf[i, pl.ds(16 * j, 16)] = z16
    rps = npad // _NS
    roff = c * npad

    def zstep(k, carry):
        pltpu.sync_copy(zbuf, acc_sh.at[pl.ds(s * rps + k * 32, 32)])
        return carry

    lax.fori_loop(0, rps // 32, zstep, 0)
    plsc.subcore_barrier()

    ebase = s * eps
    pch = 2 * _CH

    def issue_idx(i, slot):
        base = ebase + i * pch
        pltpu.async_copy(src_hbm.at[pl.ds(base, _CH)], sidxa.at[slot],
                         isems[slot])
        pltpu.async_copy(src_hbm.at[pl.ds(base + _CH, _CH)], sidxb.at[slot],
                         isems[slot])
        pltpu.async_copy(dst_hbm.at[pl.ds(base, _CH)], didxa.at[slot],
                         isems[slot])
        pltpu.async_copy(dst_hbm.at[pl.ds(base + _CH, _CH)], didxb.at[slot],
                         isems[slot])

    def wait_idx(slot):
        for ref in (sidxa, sidxb, didxa, didxb):
            pltpu.make_async_copy(src_hbm.at[pl.ds(0, _CH)], ref.at[slot],
                                  isems[slot]).wait()

    hview = h_hbm.at[pl.ds(roff, npad)]

    def fix_src(slot):
        pass

    def issue_gather(slot, rb):
        pltpu.async_copy(hview.at[sidxa.at[slot]], rowsa.at[rb], gsems[rb])
        pltpu.async_copy(hview.at[sidxb.at[slot]], rowsb.at[rb], gsems[rb])

    def wait_gather(rb):
        pltpu.make_async_copy(h_hbm.at[pl.ds(0, _CH)], rowsa.at[rb],
                              gsems[rb]).wait()
        pltpu.make_async_copy(h_hbm.at[pl.ds(0, _CH)], rowsb.at[rb],
                              gsems[rb]).wait()

    def scatter(slot, rb):
        pltpu.async_copy(rowsa.at[rb], acc_sh.at[didxa.at[slot]], ssems[rb],
                         add=True)
        pltpu.async_copy(rowsb.at[rb], acc_sh.at[didxb.at[slot]], ssems[rb],
                         add=True)

    def wait_scatter(rb):
        pltpu.make_async_copy(rowsa.at[rb], acc_sh.at[pl.ds(0, _CH)],
                              ssems[rb]).wait()
        pltpu.make_async_copy(rowsb.at[rb], acc_sh.at[pl.ds(0, _CH)],
                              ssems[rb]).wait()

    issue_idx(0, 0)
    issue_idx(1, 1)
    wait_idx(0)
    fix_src(0)
    issue_gather(0, 0)
    issue_idx(2, 2)
    wait_idx(1)
    fix_src(1)
    wait_gather(0)
    issue_gather(1, 1)
    scatter(0, 0)

    def quad(j, carry):
        for v in range(4):
            u = (1 + v) % 4
            issue_idx(4 * j + 1 + v + 2, (u + 2) % 4)
            wait_idx((u + 1) % 4)
            fix_src((u + 1) % 4)
            wait_gather(u % 2)
            wait_scatter((u + 1) % 2)
            issue_gather((u + 1) % 4, (u + 1) % 2)
            scatter(u, u % 2)
        return carry

    nmain = 4 * ((steps - 3) // 4)
    lax.fori_loop(0, nmain // 4, quad, 0)
    for t in range(nmain + 1, steps):
        u = t % 4
        if t + 2 < steps:
            issue_idx(t + 2, (u + 2) % 4)
        if t + 1 < steps:
            wait_idx((u + 1) % 4)
            fix_src((u + 1) % 4)
        wait_gather(u % 2)
        wait_scatter((u + 1) % 2)
        if t + 1 < steps:
            issue_gather((u + 1) % 4, (u + 1) % 2)
        scatter(u, u % 2)
    wait_scatter((steps - 1) % 2)

    plsc.subcore_barrier()
    pltpu.sync_copy(acc_sh.at[pl.ds(s * rps, rps)],
                    out_hbm.at[pl.ds(roff + s * rps, rps)])


def _make_conv(npad, ee):
    eps = ee // _NS
    steps = eps // (2 * _CH)
    mesh = plsc.VectorSubcoreMesh(core_axis_name="c", subcore_axis_name="s")
    return functools.partial(
        pl.kernel,
        functools.partial(_conv_body, npad, eps, steps),
        mesh=mesh,
        out_type=[jax.ShapeDtypeStruct((2 * npad, 128), jnp.float32)],
        scratch_types=[
            pltpu.VMEM((4, _CH), jnp.int32),
            pltpu.VMEM((4, _CH), jnp.int32),
            pltpu.VMEM((4, _CH), jnp.int32),
            pltpu.VMEM((4, _CH), jnp.int32),
            pltpu.VMEM((2, _CH, 128), jnp.float32),
            pltpu.VMEM((2, _CH, 128), jnp.float32),
            pltpu.VMEM((32, 128), jnp.float32),
            pltpu.VMEM_SHARED((npad, 128), jnp.float32),
            pltpu.SemaphoreType.DMA,
            pltpu.SemaphoreType.DMA,
            pltpu.SemaphoreType.DMA,
            pltpu.SemaphoreType.DMA,
            pltpu.SemaphoreType.DMA,
            pltpu.SemaphoreType.DMA,
            pltpu.SemaphoreType.DMA,
            pltpu.SemaphoreType.DMA,
        ],
    )()


def _dis_of(degq_ref):
    deg = degq_ref[0, :, 0:1] + degq_ref[1, :, 0:1] + 1.0
    return lax.rsqrt(deg)


def _tc1_body(flat_ref, spat_ref, degq_ref, wt1_ref, bt1_ref, wt2_ref,
              bt2_ref, wg1_ref, tfeat_ref, h_ref):
    dis = _dis_of(degq_ref)
    tf = jnp.maximum(
        jnp.dot(flat_ref[...], wt1_ref[...],
                preferred_element_type=jnp.float32) + bt1_ref[...], 0.0)
    tfeat_ref[...] = jnp.dot(tf, wt2_ref[...],
                             preferred_element_type=jnp.float32) + bt2_ref[...]
    h1 = jnp.dot(spat_ref[...], wg1_ref[...],
                 preferred_element_type=jnp.float32) * dis
    h_ref[0] = h1[:, :128]
    h_ref[1] = h1[:, 128:]


def _tc2_body(acc_ref, hp_ref, degq_ref, bg1_ref, wg2_ref, h_ref):
    dis = _dis_of(degq_ref)
    agg = jnp.concatenate([acc_ref[0] + hp_ref[0], acc_ref[1] + hp_ref[1]],
                          axis=1)
    x = jnp.maximum(agg * dis + bg1_ref[...], 0.0)
    h2 = jnp.dot(x, wg2_ref[...], preferred_element_type=jnp.float32) * dis
    h_ref[0] = h2[:, :128]
    h_ref[1] = h2[:, 128:]


def _tc3_body(acc_ref, hp_ref, degq_ref, tfeat_ref, bg2_ref, wsp_ref,
              bsp_ref, wa_ref, ba_ref, va_ref, wc1_ref, bc1_ref, wc2_ref,
              bc2_ref, out_ref):
    dis = _dis_of(degq_ref)
    agg = jnp.concatenate([acc_ref[0] + hp_ref[0], acc_ref[1] + hp_ref[1]],
                          axis=1)
    x2 = agg * dis + bg2_ref[...]
    sf = jnp.maximum(
        jnp.dot(x2, wsp_ref[...], preferred_element_type=jnp.float32)
        + bsp_ref[...], 0.0)
    tf = tfeat_ref[...]
    wa = wa_ref[...]
    ba = ba_ref[...]
    va = va_ref[...]
    et = jnp.dot(jnp.tanh(jnp.dot(tf, wa, preferred_element_type=jnp.float32)
                          + ba), va, preferred_element_type=jnp.float32)
    es = jnp.dot(jnp.tanh(jnp.dot(sf, wa, preferred_element_type=jnp.float32)
                          + ba), va, preferred_element_type=jnp.float32)
    a = jax.nn.sigmoid(et - es)
    fused = a * tf + (1.0 - a) * sf
    hc = jnp.maximum(
        jnp.dot(fused, wc1_ref[...], preferred_element_type=jnp.float32)
        + bc1_ref[...], 0.0)
    out_ref[...] = (jnp.dot(hc, wc2_ref[...],
                            preferred_element_type=jnp.float32) + bc2_ref[...])


def _row_spec(rb, cols):
    return pl.BlockSpec((rb, cols), lambda i: (i, 0))


def _plane_spec(rb, cols):
    return pl.BlockSpec((2, rb, cols), lambda i: (0, i, 0))


def _full_spec(shape):
    nd = len(shape)
    return pl.BlockSpec(shape, lambda i, _n=nd: (0,) * _n)


def kernel(temporal_input, spatial_input, edge_index, Wt1, bt1, Wt2, bt2,
           Wg1, bg1, Wg2, bg2, Wsp, bsp, Wa, ba, va, Wc1, bc1, Wc2, bc2):
    nn = spatial_input.shape[0]
    ee = edge_index.shape[1]
    hh = Wt1.shape[1]
    tin = temporal_input.shape[1] * temporal_input.shape[2]
    cc = Wc2.shape[1]
    rb = 1000
    nb = nn // rb
    npad = ((nn + 511) // 512) * 512

    flat = temporal_input.reshape(nn, tin)
    src = edge_index[0]
    dst = edge_index[1]

    degq = _make_deg(npad, ee)(dst)
    if isinstance(degq, (list, tuple)):
        degq = degq[0]
    degq = degq.reshape(2, npad, 128)[:, :, :8]

    tc1 = pl.pallas_call(
        _tc1_body,
        grid=(nb,),
        in_specs=[
            _row_spec(rb, tin),
            _row_spec(rb, Wg1.shape[0]),
            _plane_spec(rb, 8),
            _full_spec((tin, hh)),
            _full_spec((1, hh)),
            _full_spec((hh, hh)),
            _full_spec((1, hh)),
            _full_spec((Wg1.shape[0], hh)),
        ],
        out_specs=[_row_spec(rb, hh), _plane_spec(rb, hh // 2)],
        out_shape=[
            jax.ShapeDtypeStruct((nn, hh), jnp.float32),
            jax.ShapeDtypeStruct((2, npad, hh // 2), jnp.float32),
        ],
    )
    tfeat, h1p = tc1(flat, spatial_input, degq, Wt1, bt1.reshape(1, hh),
                     Wt2, bt2.reshape(1, hh), Wg1)

    conv = _make_conv(npad, ee)

    def _run_conv(hp):
        acc = conv(hp.reshape(2 * npad, hh // 2), src, dst)
        if isinstance(acc, (list, tuple)):
            acc = acc[0]
        return acc.reshape(2, npad, hh // 2)

    acc1 = _run_conv(h1p)

    tc2 = pl.pallas_call(
        _tc2_body,
        grid=(nb,),
        in_specs=[
            _plane_spec(rb, hh // 2),
            _plane_spec(rb, hh // 2),
            _plane_spec(rb, 8),
            _full_spec((1, hh)),
            _full_spec((hh, hh)),
        ],
        out_specs=[_plane_spec(rb, hh // 2)],
        out_shape=[jax.ShapeDtypeStruct((2, npad, hh // 2), jnp.float32)],
    )
    (h2p,) = tc2(acc1, h1p, degq, bg1.reshape(1, hh), Wg2)

    acc2 = _run_conv(h2p)

    tc3 = pl.pallas_call(
        _tc3_body,
        grid=(nb,),
        in_specs=[
            _plane_spec(rb, hh // 2),
            _plane_spec(rb, hh // 2),
            _plane_spec(rb, 8),
            _row_spec(rb, hh),
            _full_spec((1, hh)),
            _full_spec((hh, hh)),
            _full_spec((1, hh)),
            _full_spec((hh, hh)),
            _full_spec((1, hh)),
            _full_spec((hh, 1)),
            _full_spec((hh, hh // 2)),
            _full_spec((1, hh // 2)),
            _full_spec((hh // 2, cc)),
            _full_spec((1, cc)),
        ],
        out_specs=[_row_spec(rb, cc)],
        out_shape=[jax.ShapeDtypeStruct((nn, cc), jnp.float32)],
    )
    (logits,) = tc3(acc2, h2p, degq, tfeat, bg2.reshape(1, hh), Wsp,
                    bsp.reshape(1, hh), Wa, ba.reshape(1, hh),
                    va.reshape(hh, 1), Wc1, bc1.reshape(1, hh // 2), Wc2,
                    bc2.reshape(1, cc))
    return logits


# speedup vs baseline: 1.1136x; 1.1136x over previous
import functools

import jax
import jax.numpy as jnp
from jax import lax
from jax.experimental import pallas as pl
from jax.experimental.pallas import tpu as pltpu
from jax.experimental.pallas import tpu_sc as plsc

_NC = 2
_NS = 16
_CH = 80


def _deg_body(npad, depw, dsteps, dst_hbm, out_hbm, didx, ones_v, zb, deg_sh,
              isem0, isem1, isem2, isem3):
    c = lax.axis_index("c")
    s = lax.axis_index("s")
    isems = (isem0, isem1, isem2, isem3)
    one16 = jnp.ones((16,), jnp.float32)
    z16 = jnp.zeros((16,), jnp.float32)
    for i in range(_CH):
        for j in range(8):
            ones_v[i, pl.ds(16 * j, 16)] = one16
    for i in range(32):
        for j in range(8):
            zb[i, pl.ds(16 * j, 16)] = z16
    rps = npad // _NS

    def zstep(k, carry):
        pltpu.sync_copy(zb, deg_sh.at[pl.ds(s * rps + k * 32, 32)])
        return carry

    lax.fori_loop(0, rps // 32, zstep, 0)
    plsc.subcore_barrier()
    wid = s * _NC + c
    ebase = wid * depw

    def issue_idx(i, slot):
        pltpu.async_copy(dst_hbm.at[pl.ds(ebase + i * _CH, _CH)],
                         didx.at[slot], isems[slot])

    def wait_idx(slot):
        pltpu.make_async_copy(dst_hbm.at[pl.ds(0, _CH)], didx.at[slot],
                              isems[slot]).wait()

    def scatter(slot):
        pltpu.sync_copy(ones_v, deg_sh.at[didx.at[slot]], add=True)

    issue_idx(0, 0)
    issue_idx(1, 1)

    def quad(j, carry):
        for u in range(4):
            issue_idx(4 * j + u + 2, (u + 2) % 4)
            wait_idx(u)
            scatter(u)
        return carry

    nmain = 4 * ((dsteps - 2) // 4)
    lax.fori_loop(0, nmain // 4, quad, 0)
    for t in range(nmain, dsteps):
        if t + 2 < dsteps:
            issue_idx(t + 2, (t + 2) % 4)
        wait_idx(t % 4)
        scatter(t % 4)

    plsc.subcore_barrier()
    pltpu.sync_copy(deg_sh.at[pl.ds(s * rps, rps)],
                    out_hbm.at[pl.ds(c * npad + s * rps, rps)])


def _make_deg(npad, ee):
    depw = ee // (_NC * _NS)
    dsteps = depw // _CH
    mesh = plsc.VectorSubcoreMesh(core_axis_name="c", subcore_axis_name="s")
    return functools.partial(
        pl.kernel,
        functools.partial(_deg_body, npad, depw, dsteps),
        mesh=mesh,
        out_type=[jax.ShapeDtypeStruct((2 * npad, 128), jnp.float32)],
        scratch_types=[
            pltpu.VMEM((4, _CH), jnp.int32),
            pltpu.VMEM((_CH, 128), jnp.float32),
            pltpu.VMEM((32, 128), jnp.float32),
            pltpu.VMEM_SHARED((npad, 128), jnp.float32),
            pltpu.SemaphoreType.DMA,
            pltpu.SemaphoreType.DMA,
            pltpu.SemaphoreType.DMA,
            pltpu.SemaphoreType.DMA,
        ],
    )()


def _conv_body(npad, eps, steps, h_hbm, src_hbm, dst_hbm, out_hbm,
               sidxa, sidxb, didxa, didxb, rowsa, rowsb, zbuf, acc_sh,
               isem0, isem1, isem2, isem3, gsem0, gsem1, ssem0, ssem1):
    c = lax.axis_index("c")
    s = lax.axis_index("s")
    isems = (isem0, isem1, isem2, isem3)
    gsems = (gsem0, gsem1)
    ssems = (ssem0, ssem1)
    z16 = jnp.zeros((16,), jnp.float32)
    for i in range(32):
        for j in range(8):
            zbuf[i, pl.ds(16 * j, 16)] = z16
    rps = npad // _NS
    roff = c * npad

    def zstep(k, carry):
        pltpu.sync_copy(zbuf, acc_sh.at[pl.ds(s * rps + k * 32, 32)])
        return carry

    lax.fori_loop(0, rps // 32, zstep, 0)
    plsc.subcore_barrier()

    ebase = s * eps
    pch = 2 * _CH

    def issue_idx(i, slot):
        base = ebase + i * pch
        pltpu.async_copy(src_hbm.at[pl.ds(base, _CH)], sidxa.at[slot],
                         isems[slot])
        pltpu.async_copy(src_hbm.at[pl.ds(base + _CH, _CH)], sidxb.at[slot],
                         isems[slot])
        pltpu.async_copy(dst_hbm.at[pl.ds(base, _CH)], didxa.at[slot],
                         isems[slot])
        pltpu.async_copy(dst_hbm.at[pl.ds(base + _CH, _CH)], didxb.at[slot],
                         isems[slot])

    def wait_idx(slot):
        for ref in (sidxa, sidxb, didxa, didxb):
            pltpu.make_async_copy(src_hbm.at[pl.ds(0, _CH)], ref.at[slot],
                                  isems[slot]).wait()

    hview = h_hbm.at[pl.ds(roff, npad)]

    def fix_src(slot):
        pass

    def issue_gather(slot, rb):
        pltpu.async_copy(hview.at[sidxa.at[slot]], rowsa.at[rb], gsems[rb])
        pltpu.async_copy(hview.at[sidxb.at[slot]], rowsb.at[rb], gsems[rb])

    def wait_gather(rb):
        pltpu.make_async_copy(h_hbm.at[pl.ds(0, _CH)], rowsa.at[rb],
                              gsems[rb]).wait()
        pltpu.make_async_copy(h_hbm.at[pl.ds(0, _CH)], rowsb.at[rb],
                              gsems[rb]).wait()

    def scatter(slot, rb):
        pltpu.async_copy(rowsa.at[rb], acc_sh.at[didxa.at[slot]], ssems[rb],
                         add=True)
        pltpu.async_copy(rowsb.at[rb], acc_sh.at[didxb.at[slot]], ssems[rb],
                         add=True)

    def wait_scatter(rb):
        pltpu.make_async_copy(rowsa.at[rb], acc_sh.at[pl.ds(0, _CH)],
                              ssems[rb]).wait()
        pltpu.make_async_copy(rowsb.at[rb], acc_sh.at[pl.ds(0, _CH)],
                              ssems[rb]).wait()

    issue_idx(0, 0)
    issue_idx(1, 1)
    wait_idx(0)
    fix_src(0)
    issue_gather(0, 0)
    issue_idx(2, 2)
    wait_idx(1)
    issue_gather(1, 1)
    wait_gather(0)
    scatter(0, 0)

    def quad(j, carry):
        for v in range(4):
            u = (1 + v) % 4
            issue_idx(4 * j + 1 + v + 2, (u + 2) % 4)
            wait_idx((u + 1) % 4)
            wait_scatter((u + 1) % 2)
            issue_gather((u + 1) % 4, (u + 1) % 2)
            wait_gather(u % 2)
            scatter(u, u % 2)
        return carry

    nmain = 4 * ((steps - 3) // 4)
    lax.fori_loop(0, nmain // 4, quad, 0)
    for t in range(nmain + 1, steps):
        u = t % 4
        if t + 2 < steps:
            issue_idx(t + 2, (u + 2) % 4)
        if t + 1 < steps:
            wait_idx((u + 1) % 4)
            wait_scatter((u + 1) % 2)
            issue_gather((u + 1) % 4, (u + 1) % 2)
        wait_gather(u % 2)
        scatter(u, u % 2)
    wait_scatter((steps - 2) % 2)
    wait_scatter((steps - 1) % 2)

    plsc.subcore_barrier()
    pltpu.sync_copy(acc_sh.at[pl.ds(s * rps, rps)],
                    out_hbm.at[pl.ds(roff + s * rps, rps)])


def _make_conv(npad, ee):
    eps = ee // _NS
    steps = eps // (2 * _CH)
    mesh = plsc.VectorSubcoreMesh(core_axis_name="c", subcore_axis_name="s")
    return functools.partial(
        pl.kernel,
        functools.partial(_conv_body, npad, eps, steps),
        mesh=mesh,
        out_type=[jax.ShapeDtypeStruct((2 * npad, 128), jnp.float32)],
        scratch_types=[
            pltpu.VMEM((4, _CH), jnp.int32),
            pltpu.VMEM((4, _CH), jnp.int32),
            pltpu.VMEM((4, _CH), jnp.int32),
            pltpu.VMEM((4, _CH), jnp.int32),
            pltpu.VMEM((2, _CH, 128), jnp.float32),
            pltpu.VMEM((2, _CH, 128), jnp.float32),
            pltpu.VMEM((32, 128), jnp.float32),
            pltpu.VMEM_SHARED((npad, 128), jnp.float32),
            pltpu.SemaphoreType.DMA,
            pltpu.SemaphoreType.DMA,
            pltpu.SemaphoreType.DMA,
            pltpu.SemaphoreType.DMA,
            pltpu.SemaphoreType.DMA,
            pltpu.SemaphoreType.DMA,
            pltpu.SemaphoreType.DMA,
            pltpu.SemaphoreType.DMA,
        ],
    )()


def _dis_of(degq_ref):
    deg = degq_ref[0, :, 0:1] + degq_ref[1, :, 0:1] + 1.0
    return lax.rsqrt(deg)


def _tc1_body(flat_ref, spat_ref, degq_ref, wt1_ref, bt1_ref, wt2_ref,
              bt2_ref, wg1_ref, tfeat_ref, h_ref):
    dis = _dis_of(degq_ref)
    tf = jnp.maximum(
        jnp.dot(flat_ref[...], wt1_ref[...],
                preferred_element_type=jnp.float32) + bt1_ref[...], 0.0)
    tfeat_ref[...] = jnp.dot(tf, wt2_ref[...],
                             preferred_element_type=jnp.float32) + bt2_ref[...]
    h1 = jnp.dot(spat_ref[...], wg1_ref[...],
                 preferred_element_type=jnp.float32) * dis
    h_ref[0] = h1[:, :128]
    h_ref[1] = h1[:, 128:]


def _tc2_body(acc_ref, hp_ref, degq_ref, bg1_ref, wg2_ref, h_ref):
    dis = _dis_of(degq_ref)
    agg = jnp.concatenate([acc_ref[0] + hp_ref[0], acc_ref[1] + hp_ref[1]],
                          axis=1)
    x = jnp.maximum(agg * dis + bg1_ref[...], 0.0)
    h2 = jnp.dot(x, wg2_ref[...], preferred_element_type=jnp.float32) * dis
    h_ref[0] = h2[:, :128]
    h_ref[1] = h2[:, 128:]


def _tc3_body(acc_ref, hp_ref, degq_ref, tfeat_ref, bg2_ref, wsp_ref,
              bsp_ref, wa_ref, ba_ref, va_ref, wc1_ref, bc1_ref, wc2_ref,
              bc2_ref, out_ref):
    dis = _dis_of(degq_ref)
    agg = jnp.concatenate([acc_ref[0] + hp_ref[0], acc_ref[1] + hp_ref[1]],
                          axis=1)
    x2 = agg * dis + bg2_ref[...]
    sf = jnp.maximum(
        jnp.dot(x2, wsp_ref[...], preferred_element_type=jnp.float32)
        + bsp_ref[...], 0.0)
    tf = tfeat_ref[...]
    wa = wa_ref[...]
    ba = ba_ref[...]
    va = va_ref[...]
    et = jnp.dot(jnp.tanh(jnp.dot(tf, wa, preferred_element_type=jnp.float32)
                          + ba), va, preferred_element_type=jnp.float32)
    es = jnp.dot(jnp.tanh(jnp.dot(sf, wa, preferred_element_type=jnp.float32)
                          + ba), va, preferred_element_type=jnp.float32)
    a = jax.nn.sigmoid(et - es)
    fused = a * tf + (1.0 - a) * sf
    hc = jnp.maximum(
        jnp.dot(fused, wc1_ref[...], preferred_element_type=jnp.float32)
        + bc1_ref[...], 0.0)
    out_ref[...] = (jnp.dot(hc, wc2_ref[...],
                            preferred_element_type=jnp.float32) + bc2_ref[...])


def _row_spec(rb, cols):
    return pl.BlockSpec((rb, cols), lambda i: (i, 0))


def _plane_spec(rb, cols):
    return pl.BlockSpec((2, rb, cols), lambda i: (0, i, 0))


def _full_spec(shape):
    nd = len(shape)
    return pl.BlockSpec(shape, lambda i, _n=nd: (0,) * _n)


def kernel(temporal_input, spatial_input, edge_index, Wt1, bt1, Wt2, bt2,
           Wg1, bg1, Wg2, bg2, Wsp, bsp, Wa, ba, va, Wc1, bc1, Wc2, bc2):
    nn = spatial_input.shape[0]
    ee = edge_index.shape[1]
    hh = Wt1.shape[1]
    tin = temporal_input.shape[1] * temporal_input.shape[2]
    cc = Wc2.shape[1]
    rb = 1000
    nb = nn // rb
    npad = ((nn + 511) // 512) * 512

    flat = temporal_input.reshape(nn, tin)
    src = edge_index[0]
    dst = edge_index[1]

    degq = _make_deg(npad, ee)(dst)
    if isinstance(degq, (list, tuple)):
        degq = degq[0]
    degq = degq.reshape(2, npad, 128)[:, :, :8]

    tc1 = pl.pallas_call(
        _tc1_body,
        grid=(nb,),
        in_specs=[
            _row_spec(rb, tin),
            _row_spec(rb, Wg1.shape[0]),
            _plane_spec(rb, 8),
            _full_spec((tin, hh)),
            _full_spec((1, hh)),
            _full_spec((hh, hh)),
            _full_spec((1, hh)),
            _full_spec((Wg1.shape[0], hh)),
        ],
        out_specs=[_row_spec(rb, hh), _plane_spec(rb, hh // 2)],
        out_shape=[
            jax.ShapeDtypeStruct((nn, hh), jnp.float32),
            jax.ShapeDtypeStruct((2, npad, hh // 2), jnp.float32),
        ],
    )
    tfeat, h1p = tc1(flat, spatial_input, degq, Wt1, bt1.reshape(1, hh),
                     Wt2, bt2.reshape(1, hh), Wg1)

    conv = _make_conv(npad, ee)

    def _run_conv(hp):
        acc = conv(hp.reshape(2 * npad, hh // 2), src, dst)
        if isinstance(acc, (list, tuple)):
            acc = acc[0]
        return acc.reshape(2, npad, hh // 2)

    acc1 = _run_conv(h1p)

    tc2 = pl.pallas_call(
        _tc2_body,
        grid=(nb,),
        in_specs=[
            _plane_spec(rb, hh // 2),
            _plane_spec(rb, hh // 2),
            _plane_spec(rb, 8),
            _full_spec((1, hh)),
            _full_spec((hh, hh)),
        ],
        out_specs=[_plane_spec(rb, hh // 2)],
        out_shape=[jax.ShapeDtypeStruct((2, npad, hh // 2), jnp.float32)],
    )
    (h2p,) = tc2(acc1, h1p, degq, bg1.reshape(1, hh), Wg2)

    acc2 = _run_conv(h2p)

    tc3 = pl.pallas_call(
        _tc3_body,
        grid=(nb,),
        in_specs=[
            _plane_spec(rb, hh // 2),
            _plane_spec(rb, hh // 2),
            _plane_spec(rb, 8),
            _row_spec(rb, hh),
            _full_spec((1, hh)),
            _full_spec((hh, hh)),
            _full_spec((1, hh)),
            _full_spec((hh, hh)),
            _full_spec((1, hh)),
            _full_spec((hh, 1)),
            _full_spec((hh, hh // 2)),
            _full_spec((1, hh // 2)),
            _full_spec((hh // 2, cc)),
            _full_spec((1, cc)),
        ],
        out_specs=[_row_spec(rb, cc)],
        out_shape=[jax.ShapeDtypeStruct((nn, cc), jnp.float32)],
    )
    (logits,) = tc3(acc2, h2p, degq, tfeat, bg2.reshape(1, hh), Wsp,
                    bsp.reshape(1, hh), Wa, ba.reshape(1, hh),
                    va.reshape(hh, 1), Wc1, bc1.reshape(1, hh // 2), Wc2,
                    bc2.reshape(1, cc))
    return logits


# speedup vs baseline: 1.1841x; 1.0634x over previous
import functools

import jax
import jax.numpy as jnp
from jax import lax
from jax.experimental import pallas as pl
from jax.experimental.pallas import tpu as pltpu
from jax.experimental.pallas import tpu_sc as plsc

_NC = 2
_NS = 16
_CH = 80


def _deg_body(npad, depw, dsteps, dst_hbm, out_hbm, didx, ones_v, zb, vals,
              deg_sh, isem0, isem1, isem2, isem3):
    c = lax.axis_index("c")
    s = lax.axis_index("s")
    isems = (isem0, isem1, isem2, isem3)
    one16 = jnp.ones((16,), jnp.float32)
    z16 = jnp.zeros((16,), jnp.float32)
    for i in range(_CH // 16):
        ones_v[pl.ds(16 * i, 16)] = one16
    rps = npad // _NS
    for i in range(rps // 16):
        zb[pl.ds(16 * i, 16)] = z16
    deg_view = deg_sh.at[pl.ds(s * npad, npad)]

    def zstep(k, carry):
        pltpu.sync_copy(zb, deg_view.at[pl.ds(k * rps, rps)])
        return carry

    lax.fori_loop(0, _NS, zstep, 0)
    plsc.subcore_barrier()
    wid = s * _NC + c
    ebase = wid * depw

    def issue_idx(i, slot):
        pltpu.async_copy(dst_hbm.at[pl.ds(ebase + i * _CH, _CH)],
                         didx.at[slot], isems[slot])

    def wait_idx(slot):
        pltpu.make_async_copy(dst_hbm.at[pl.ds(0, _CH)], didx.at[slot],
                              isems[slot]).wait()

    def scatter(slot):
        pltpu.sync_copy(ones_v, deg_view.at[didx.at[slot]], add=True)

    issue_idx(0, 0)
    issue_idx(1, 1)

    def quad(j, carry):
        for u in range(4):
            issue_idx(4 * j + u + 2, (u + 2) % 4)
            wait_idx(u)
            scatter(u)
        return carry

    nmain = 4 * ((dsteps - 2) // 4)
    lax.fori_loop(0, nmain // 4, quad, 0)
    for t in range(nmain, dsteps):
        if t + 2 < dsteps:
            issue_idx(t + 2, (t + 2) % 4)
        wait_idx(t % 4)
        scatter(t % 4)

    plsc.subcore_barrier()
    srow = s * rps
    for r in range(_NS):
        pltpu.async_copy(deg_sh.at[pl.ds(r * npad + srow, rps)], vals.at[r],
                         isems[0])
    for r in range(_NS):
        pltpu.make_async_copy(deg_sh.at[pl.ds(0, rps)], vals.at[r],
                              isems[0]).wait()
    for g in range(rps // 16):
        sl = pl.ds(16 * g, 16)
        v = vals[0, sl]
        for r in range(1, _NS):
            v = v + vals[r, sl]
        vals[0, sl] = v
    pltpu.sync_copy(vals.at[0],
                    out_hbm.at[pl.ds(c * npad + srow, rps)])


def _make_deg(npad, ee):
    depw = ee // (_NC * _NS)
    dsteps = depw // _CH
    mesh = plsc.VectorSubcoreMesh(core_axis_name="c", subcore_axis_name="s")
    return functools.partial(
        pl.kernel,
        functools.partial(_deg_body, npad, depw, dsteps),
        mesh=mesh,
        out_type=[jax.ShapeDtypeStruct((2 * npad,), jnp.float32)],
        scratch_types=[
            pltpu.VMEM((4, _CH), jnp.int32),
            pltpu.VMEM((_CH,), jnp.float32),
            pltpu.VMEM((npad // _NS,), jnp.float32),
            pltpu.VMEM((_NS, npad // _NS), jnp.float32),
            pltpu.VMEM_SHARED((_NS * npad,), jnp.float32),
            pltpu.SemaphoreType.DMA,
            pltpu.SemaphoreType.DMA,
            pltpu.SemaphoreType.DMA,
            pltpu.SemaphoreType.DMA,
        ],
    )()


def _conv_body(npad, eps, steps, h_hbm, src_hbm, dst_hbm, out_hbm,
               sidxa, sidxb, didxa, didxb, rowsa, rowsb, zbuf, acc_sh,
               isem0, isem1, isem2, isem3, gsem0, gsem1, ssem0, ssem1):
    c = lax.axis_index("c")
    s = lax.axis_index("s")
    isems = (isem0, isem1, isem2, isem3)
    gsems = (gsem0, gsem1)
    ssems = (ssem0, ssem1)
    z16 = jnp.zeros((16,), jnp.float32)
    for i in range(32):
        for j in range(8):
            zbuf[i, pl.ds(16 * j, 16)] = z16
    rps = npad // _NS
    roff = c * npad

    def zstep(k, carry):
        pltpu.sync_copy(zbuf, acc_sh.at[pl.ds(s * rps + k * 32, 32)])
        return carry

    lax.fori_loop(0, rps // 32, zstep, 0)
    plsc.subcore_barrier()

    ebase = s * eps
    pch = 2 * _CH

    def issue_idx(i, slot):
        base = ebase + i * pch
        pltpu.async_copy(src_hbm.at[pl.ds(base, _CH)], sidxa.at[slot],
                         isems[slot])
        pltpu.async_copy(src_hbm.at[pl.ds(base + _CH, _CH)], sidxb.at[slot],
                         isems[slot])
        pltpu.async_copy(dst_hbm.at[pl.ds(base, _CH)], didxa.at[slot],
                         isems[slot])
        pltpu.async_copy(dst_hbm.at[pl.ds(base + _CH, _CH)], didxb.at[slot],
                         isems[slot])

    def wait_idx(slot):
        for ref in (sidxa, sidxb, didxa, didxb):
            pltpu.make_async_copy(src_hbm.at[pl.ds(0, _CH)], ref.at[slot],
                                  isems[slot]).wait()

    hview = h_hbm.at[pl.ds(roff, npad)]

    def fix_src(slot):
        pass

    def issue_gather(slot, rb):
        pltpu.async_copy(hview.at[sidxa.at[slot]], rowsa.at[rb], gsems[rb])
        pltpu.async_copy(hview.at[sidxb.at[slot]], rowsb.at[rb], gsems[rb])

    def wait_gather(rb):
        pltpu.make_async_copy(h_hbm.at[pl.ds(0, _CH)], rowsa.at[rb],
                              gsems[rb]).wait()
        pltpu.make_async_copy(h_hbm.at[pl.ds(0, _CH)], rowsb.at[rb],
                              gsems[rb]).wait()

    def scatter(slot, rb):
        pltpu.async_copy(rowsa.at[rb], acc_sh.at[didxa.at[slot]], ssems[rb],
                         add=True)
        pltpu.async_copy(rowsb.at[rb], acc_sh.at[didxb.at[slot]], ssems[rb],
                         add=True)

    def wait_scatter(rb):
        pltpu.make_async_copy(rowsa.at[rb], acc_sh.at[pl.ds(0, _CH)],
                              ssems[rb]).wait()
        pltpu.make_async_copy(rowsb.at[rb], acc_sh.at[pl.ds(0, _CH)],
                              ssems[rb]).wait()

    issue_idx(0, 0)
    issue_idx(1, 1)
    wait_idx(0)
    fix_src(0)
    issue_gather(0, 0)
    issue_idx(2, 2)
    wait_idx(1)
    issue_gather(1, 1)
    wait_gather(0)
    scatter(0, 0)

    def quad(j, carry):
        for v in range(4):
            u = (1 + v) % 4
            issue_idx(4 * j + 1 + v + 2, (u + 2) % 4)
            wait_idx((u + 1) % 4)
            wait_scatter((u + 1) % 2)
            issue_gather((u + 1) % 4, (u + 1) % 2)
            wait_gather(u % 2)
            scatter(u, u % 2)
        return carry

    nmain = 4 * ((steps - 3) // 4)
    lax.fori_loop(0, nmain // 4, quad, 0)
    for t in range(nmain + 1, steps):
        u = t % 4
        if t + 2 < steps:
            issue_idx(t + 2, (u + 2) % 4)
        if t + 1 < steps:
            wait_idx((u + 1) % 4)
            wait_scatter((u + 1) % 2)
            issue_gather((u + 1) % 4, (u + 1) % 2)
        wait_gather(u % 2)
        scatter(u, u % 2)
    wait_scatter((steps - 2) % 2)
    wait_scatter((steps - 1) % 2)

    plsc.subcore_barrier()
    pltpu.sync_copy(acc_sh.at[pl.ds(s * rps, rps)],
                    out_hbm.at[pl.ds(roff + s * rps, rps)])


def _make_conv(npad, ee):
    eps = ee // _NS
    steps = eps // (2 * _CH)
    mesh = plsc.VectorSubcoreMesh(core_axis_name="c", subcore_axis_name="s")
    return functools.partial(
        pl.kernel,
        functools.partial(_conv_body, npad, eps, steps),
        mesh=mesh,
        out_type=[jax.ShapeDtypeStruct((2 * npad, 128), jnp.float32)],
        scratch_types=[
            pltpu.VMEM((4, _CH), jnp.int32),
            pltpu.VMEM((4, _CH), jnp.int32),
            pltpu.VMEM((4, _CH), jnp.int32),
            pltpu.VMEM((4, _CH), jnp.int32),
            pltpu.VMEM((2, _CH, 128), jnp.float32),
            pltpu.VMEM((2, _CH, 128), jnp.float32),
            pltpu.VMEM((32, 128), jnp.float32),
            pltpu.VMEM_SHARED((npad, 128), jnp.float32),
            pltpu.SemaphoreType.DMA,
            pltpu.SemaphoreType.DMA,
            pltpu.SemaphoreType.DMA,
            pltpu.SemaphoreType.DMA,
            pltpu.SemaphoreType.DMA,
            pltpu.SemaphoreType.DMA,
            pltpu.SemaphoreType.DMA,
            pltpu.SemaphoreType.DMA,
        ],
    )()


def _dis_of(d0_ref, d1_ref):
    deg = d0_ref[...] + d1_ref[...] + 1.0
    return lax.rsqrt(deg)


def _tc1_body(flat_ref, spat_ref, d0_ref, d1_ref, wt1_ref, bt1_ref, wt2_ref,
              bt2_ref, wg1_ref, tfeat_ref, h_ref):
    dis = _dis_of(d0_ref, d1_ref)
    tf = jnp.maximum(
        jnp.dot(flat_ref[...], wt1_ref[...],
                preferred_element_type=jnp.float32) + bt1_ref[...], 0.0)
    tfeat_ref[...] = jnp.dot(tf, wt2_ref[...],
                             preferred_element_type=jnp.float32) + bt2_ref[...]
    h1 = jnp.dot(spat_ref[...], wg1_ref[...],
                 preferred_element_type=jnp.float32) * dis
    h_ref[0] = h1[:, :128]
    h_ref[1] = h1[:, 128:]


def _tc2_body(acc_ref, hp_ref, d0_ref, d1_ref, bg1_ref, wg2_ref, h_ref):
    dis = _dis_of(d0_ref, d1_ref)
    agg = jnp.concatenate([acc_ref[0] + hp_ref[0], acc_ref[1] + hp_ref[1]],
                          axis=1)
    x = jnp.maximum(agg * dis + bg1_ref[...], 0.0)
    h2 = jnp.dot(x, wg2_ref[...], preferred_element_type=jnp.float32) * dis
    h_ref[0] = h2[:, :128]
    h_ref[1] = h2[:, 128:]


def _tc3_body(acc_ref, hp_ref, d0_ref, d1_ref, tfeat_ref, bg2_ref, wsp_ref,
              bsp_ref, wa_ref, ba_ref, va_ref, wc1_ref, bc1_ref, wc2_ref,
              bc2_ref, out_ref):
    dis = _dis_of(d0_ref, d1_ref)
    agg = jnp.concatenate([acc_ref[0] + hp_ref[0], acc_ref[1] + hp_ref[1]],
                          axis=1)
    x2 = agg * dis + bg2_ref[...]
    sf = jnp.maximum(
        jnp.dot(x2, wsp_ref[...], preferred_element_type=jnp.float32)
        + bsp_ref[...], 0.0)
    tf = tfeat_ref[...]
    wa = wa_ref[...]
    ba = ba_ref[...]
    va = va_ref[...]
    et = jnp.dot(jnp.tanh(jnp.dot(tf, wa, preferred_element_type=jnp.float32)
                          + ba), va, preferred_element_type=jnp.float32)
    es = jnp.dot(jnp.tanh(jnp.dot(sf, wa, preferred_element_type=jnp.float32)
                          + ba), va, preferred_element_type=jnp.float32)
    a = jax.nn.sigmoid(et - es)
    fused = a * tf + (1.0 - a) * sf
    hc = jnp.maximum(
        jnp.dot(fused, wc1_ref[...], preferred_element_type=jnp.float32)
        + bc1_ref[...], 0.0)
    out_ref[...] = (jnp.dot(hc, wc2_ref[...],
                            preferred_element_type=jnp.float32) + bc2_ref[...])


def _row_spec(rb, cols):
    return pl.BlockSpec((rb, cols), lambda i: (i, 0))


def _plane_spec(rb, cols):
    return pl.BlockSpec((2, rb, cols), lambda i: (0, i, 0))


def _full_spec(shape):
    nd = len(shape)
    return pl.BlockSpec(shape, lambda i, _n=nd: (0,) * _n)


def kernel(temporal_input, spatial_input, edge_index, Wt1, bt1, Wt2, bt2,
           Wg1, bg1, Wg2, bg2, Wsp, bsp, Wa, ba, va, Wc1, bc1, Wc2, bc2):
    nn = spatial_input.shape[0]
    ee = edge_index.shape[1]
    hh = Wt1.shape[1]
    tin = temporal_input.shape[1] * temporal_input.shape[2]
    cc = Wc2.shape[1]
    rb = 1000
    nb = nn // rb
    npad = ((nn + 511) // 512) * 512

    flat = temporal_input.reshape(nn, tin)
    src = edge_index[0]
    dst = edge_index[1]

    degq = _make_deg(npad, ee)(dst)
    if isinstance(degq, (list, tuple)):
        degq = degq[0]
    deg0 = degq[:npad].reshape(npad, 1)
    deg1 = degq[npad:].reshape(npad, 1)

    tc1 = pl.pallas_call(
        _tc1_body,
        grid=(nb,),
        in_specs=[
            _row_spec(rb, tin),
            _row_spec(rb, Wg1.shape[0]),
            _row_spec(rb, 1),
            _row_spec(rb, 1),
            _full_spec((tin, hh)),
            _full_spec((1, hh)),
            _full_spec((hh, hh)),
            _full_spec((1, hh)),
            _full_spec((Wg1.shape[0], hh)),
        ],
        out_specs=[_row_spec(rb, hh), _plane_spec(rb, hh // 2)],
        out_shape=[
            jax.ShapeDtypeStruct((nn, hh), jnp.float32),
            jax.ShapeDtypeStruct((2, npad, hh // 2), jnp.float32),
        ],
    )
    tfeat, h1p = tc1(flat, spatial_input, deg0, deg1, Wt1, bt1.reshape(1, hh),
                     Wt2, bt2.reshape(1, hh), Wg1)

    conv = _make_conv(npad, ee)

    def _run_conv(hp):
        acc = conv(hp.reshape(2 * npad, hh // 2), src, dst)
        if isinstance(acc, (list, tuple)):
            acc = acc[0]
        return acc.reshape(2, npad, hh // 2)

    acc1 = _run_conv(h1p)

    tc2 = pl.pallas_call(
        _tc2_body,
        grid=(nb,),
        in_specs=[
            _plane_spec(rb, hh // 2),
            _plane_spec(rb, hh // 2),
            _row_spec(rb, 1),
            _row_spec(rb, 1),
            _full_spec((1, hh)),
            _full_spec((hh, hh)),
        ],
        out_specs=[_plane_spec(rb, hh // 2)],
        out_shape=[jax.ShapeDtypeStruct((2, npad, hh // 2), jnp.float32)],
    )
    (h2p,) = tc2(acc1, h1p, deg0, deg1, bg1.reshape(1, hh), Wg2)

    acc2 = _run_conv(h2p)

    tc3 = pl.pallas_call(
        _tc3_body,
        grid=(nb,),
        in_specs=[
            _plane_spec(rb, hh // 2),
            _plane_spec(rb, hh // 2),
            _row_spec(rb, 1),
            _row_spec(rb, 1),
            _row_spec(rb, hh),
            _full_spec((1, hh)),
            _full_spec((hh, hh)),
            _full_spec((1, hh)),
            _full_spec((hh, hh)),
            _full_spec((1, hh)),
            _full_spec((hh, 1)),
            _full_spec((hh, hh // 2)),
            _full_spec((1, hh // 2)),
            _full_spec((hh // 2, cc)),
            _full_spec((1, cc)),
        ],
        out_specs=[_row_spec(rb, cc)],
        out_shape=[jax.ShapeDtypeStruct((nn, cc), jnp.float32)],
    )
    (logits,) = tc3(acc2, h2p, deg0, deg1, tfeat, bg2.reshape(1, hh), Wsp,
                    bsp.reshape(1, hh), Wa, ba.reshape(1, hh),
                    va.reshape(hh, 1), Wc1, bc1.reshape(1, hh // 2), Wc2,
                    bc2.reshape(1, cc))
    return logits


# speedup vs baseline: 1.1928x; 1.0073x over previous
import functools

import jax
import jax.numpy as jnp
from jax import lax
from jax.experimental import pallas as pl
from jax.experimental.pallas import tpu as pltpu
from jax.experimental.pallas import tpu_sc as plsc

_NC = 2
_NS = 16
_CH = 80


def _deg_body(npad, depw, dsteps, dst_hbm, out_hbm, didx, ones_v, zb, vals,
              deg_sh, isem0, isem1, isem2, isem3):
    c = lax.axis_index("c")
    s = lax.axis_index("s")
    isems = (isem0, isem1, isem2, isem3)
    one16 = jnp.ones((16,), jnp.float32)
    z16 = jnp.zeros((16,), jnp.float32)
    for i in range(_CH // 16):
        ones_v[pl.ds(16 * i, 16)] = one16
    rps = npad // _NS
    for i in range(rps // 16):
        zb[pl.ds(16 * i, 16)] = z16
    deg_view = deg_sh.at[pl.ds(s * npad, npad)]

    def zstep(k, carry):
        pltpu.sync_copy(zb, deg_view.at[pl.ds(k * rps, rps)])
        return carry

    lax.fori_loop(0, _NS, zstep, 0)
    plsc.subcore_barrier()
    wid = s * _NC + c
    ebase = wid * depw

    def issue_idx(i, slot):
        pltpu.async_copy(dst_hbm.at[pl.ds(ebase + i * _CH, _CH)],
                         didx.at[slot], isems[slot])

    def wait_idx(slot):
        pltpu.make_async_copy(dst_hbm.at[pl.ds(0, _CH)], didx.at[slot],
                              isems[slot]).wait()

    def scatter(slot):
        pltpu.sync_copy(ones_v, deg_view.at[didx.at[slot]], add=True)

    issue_idx(0, 0)
    issue_idx(1, 1)

    def quad(j, carry):
        for u in range(4):
            issue_idx(4 * j + u + 2, (u + 2) % 4)
            wait_idx(u)
            scatter(u)
        return carry

    nmain = 4 * ((dsteps - 2) // 4)
    lax.fori_loop(0, nmain // 4, quad, 0)
    for t in range(nmain, dsteps):
        if t + 2 < dsteps:
            issue_idx(t + 2, (t + 2) % 4)
        wait_idx(t % 4)
        scatter(t % 4)

    plsc.subcore_barrier()
    srow = s * rps
    for r in range(_NS):
        pltpu.async_copy(deg_sh.at[pl.ds(r * npad + srow, rps)], vals.at[r],
                         isems[0])
    for r in range(_NS):
        pltpu.make_async_copy(deg_sh.at[pl.ds(0, rps)], vals.at[r],
                              isems[0]).wait()
    for g in range(rps // 16):
        sl = pl.ds(16 * g, 16)
        v = vals[0, sl]
        for r in range(1, _NS):
            v = v + vals[r, sl]
        vals[0, sl] = v
    pltpu.sync_copy(vals.at[0],
                    out_hbm.at[pl.ds(c * npad + srow, rps)])


def _make_deg(npad, ee):
    depw = ee // (_NC * _NS)
    dsteps = depw // _CH
    mesh = plsc.VectorSubcoreMesh(core_axis_name="c", subcore_axis_name="s")
    return functools.partial(
        pl.kernel,
        functools.partial(_deg_body, npad, depw, dsteps),
        mesh=mesh,
        out_type=[jax.ShapeDtypeStruct((2 * npad,), jnp.float32)],
        scratch_types=[
            pltpu.VMEM((4, _CH), jnp.int32),
            pltpu.VMEM((_CH,), jnp.float32),
            pltpu.VMEM((npad // _NS,), jnp.float32),
            pltpu.VMEM((_NS, npad // _NS), jnp.float32),
            pltpu.VMEM_SHARED((_NS * npad,), jnp.float32),
            pltpu.SemaphoreType.DMA,
            pltpu.SemaphoreType.DMA,
            pltpu.SemaphoreType.DMA,
            pltpu.SemaphoreType.DMA,
        ],
    )()


def _conv_body(npad, eps, steps, h_hbm, src_hbm, dst_hbm, out_hbm,
               sidxa, sidxb, didxa, didxb, rowsa, rowsb, zbuf, acc_sh,
               isem0, isem1, isem2, isem3, gsem0, gsem1, ssem0, ssem1):
    c = lax.axis_index("c")
    s = lax.axis_index("s")
    isems = (isem0, isem1, isem2, isem3)
    gsems = (gsem0, gsem1)
    ssems = (ssem0, ssem1)
    z16 = jnp.zeros((16,), jnp.float32)
    for i in range(32):
        for j in range(8):
            zbuf[i, pl.ds(16 * j, 16)] = z16
    rps = npad // _NS
    roff = c * npad

    def zstep(k, carry):
        pltpu.sync_copy(zbuf, acc_sh.at[pl.ds(s * rps + k * 32, 32)])
        return carry

    lax.fori_loop(0, rps // 32, zstep, 0)
    plsc.subcore_barrier()

    ebase = s * eps
    pch = 2 * _CH

    def issue_idx(i, slot):
        base = ebase + i * pch
        pltpu.async_copy(src_hbm.at[pl.ds(base, _CH)], sidxa.at[slot],
                         isems[slot])
        pltpu.async_copy(src_hbm.at[pl.ds(base + _CH, _CH)], sidxb.at[slot],
                         isems[slot])
        pltpu.async_copy(dst_hbm.at[pl.ds(base, _CH)], didxa.at[slot],
                         isems[slot])
        pltpu.async_copy(dst_hbm.at[pl.ds(base + _CH, _CH)], didxb.at[slot],
                         isems[slot])

    def wait_idx(slot):
        for ref in (sidxa, sidxb, didxa, didxb):
            pltpu.make_async_copy(src_hbm.at[pl.ds(0, _CH)], ref.at[slot],
                                  isems[slot]).wait()

    hview = h_hbm.at[pl.ds(roff, npad)]

    def fix_src(slot):
        pass

    def issue_gather(slot, rb):
        pltpu.async_copy(hview.at[sidxa.at[slot]], rowsa.at[rb], gsems[rb])
        pltpu.async_copy(hview.at[sidxb.at[slot]], rowsb.at[rb], gsems[rb])

    def wait_gather(rb):
        pltpu.make_async_copy(h_hbm.at[pl.ds(0, _CH)], rowsa.at[rb],
                              gsems[rb]).wait()
        pltpu.make_async_copy(h_hbm.at[pl.ds(0, _CH)], rowsb.at[rb],
                              gsems[rb]).wait()

    def scatter(slot, rb):
        pltpu.async_copy(rowsa.at[rb], acc_sh.at[didxa.at[slot]], ssems[rb],
                         add=True)
        pltpu.async_copy(rowsb.at[rb], acc_sh.at[didxb.at[slot]], ssems[rb],
                         add=True)

    def wait_scatter(rb):
        pltpu.make_async_copy(rowsa.at[rb], acc_sh.at[pl.ds(0, _CH)],
                              ssems[rb]).wait()
        pltpu.make_async_copy(rowsb.at[rb], acc_sh.at[pl.ds(0, _CH)],
                              ssems[rb]).wait()

    issue_idx(0, 0)
    issue_idx(1, 1)
    wait_idx(0)
    fix_src(0)
    issue_gather(0, 0)
    issue_idx(2, 2)
    wait_idx(1)
    issue_gather(1, 1)
    wait_gather(0)
    scatter(0, 0)

    def quad(j, carry):
        for v in range(4):
            u = (1 + v) % 4
            issue_idx(4 * j + 1 + v + 2, (u + 2) % 4)
            wait_idx((u + 1) % 4)
            wait_scatter((u + 1) % 2)
            issue_gather((u + 1) % 4, (u + 1) % 2)
            wait_gather(u % 2)
            scatter(u, u % 2)
        return carry

    nmain = 4 * ((steps - 3) // 4)
    lax.fori_loop(0, nmain // 4, quad, 0)
    for t in range(nmain + 1, steps):
        u = t % 4
        if t + 2 < steps:
            issue_idx(t + 2, (u + 2) % 4)
        if t + 1 < steps:
            wait_idx((u + 1) % 4)
            wait_scatter((u + 1) % 2)
            issue_gather((u + 1) % 4, (u + 1) % 2)
        wait_gather(u % 2)
        scatter(u, u % 2)
    wait_scatter((steps - 2) % 2)
    wait_scatter((steps - 1) % 2)

    plsc.subcore_barrier()
    pltpu.sync_copy(acc_sh.at[pl.ds(s * rps, rps)],
                    out_hbm.at[pl.ds(roff + s * rps, rps)])


def _make_conv(npad, ee):
    eps = ee // _NS
    steps = eps // (2 * _CH)
    mesh = plsc.VectorSubcoreMesh(core_axis_name="c", subcore_axis_name="s")
    return functools.partial(
        pl.kernel,
        functools.partial(_conv_body, npad, eps, steps),
        mesh=mesh,
        out_type=[jax.ShapeDtypeStruct((2 * npad, 128), jnp.float32)],
        scratch_types=[
            pltpu.VMEM((4, _CH), jnp.int32),
            pltpu.VMEM((4, _CH), jnp.int32),
            pltpu.VMEM((4, _CH), jnp.int32),
            pltpu.VMEM((4, _CH), jnp.int32),
            pltpu.VMEM((2, _CH, 128), jnp.float32),
            pltpu.VMEM((2, _CH, 128), jnp.float32),
            pltpu.VMEM((32, 128), jnp.float32),
            pltpu.VMEM_SHARED((npad, 128), jnp.float32),
            pltpu.SemaphoreType.DMA,
            pltpu.SemaphoreType.DMA,
            pltpu.SemaphoreType.DMA,
            pltpu.SemaphoreType.DMA,
            pltpu.SemaphoreType.DMA,
            pltpu.SemaphoreType.DMA,
            pltpu.SemaphoreType.DMA,
            pltpu.SemaphoreType.DMA,
        ],
    )()


def _dis_of(d0_ref, d1_ref):
    deg = d0_ref[...] + d1_ref[...] + 1.0
    return lax.rsqrt(deg)


def _tc0_body(flat_ref, wt1_ref, bt1_ref, wt2_ref, bt2_ref, tfeat_ref):
    tf = jnp.maximum(
        jnp.dot(flat_ref[...], wt1_ref[...],
                preferred_element_type=jnp.float32) + bt1_ref[...], 0.0)
    tfeat_ref[...] = jnp.dot(tf, wt2_ref[...],
                             preferred_element_type=jnp.float32) + bt2_ref[...]


def _tc1_body(spat_ref, d0_ref, d1_ref, wg1_ref, h_ref):
    dis = _dis_of(d0_ref, d1_ref)
    h1 = jnp.dot(spat_ref[...], wg1_ref[...],
                 preferred_element_type=jnp.float32) * dis
    h_ref[0] = h1[:, :128]
    h_ref[1] = h1[:, 128:]


def _tc2_body(acc_ref, hp_ref, d0_ref, d1_ref, bg1_ref, wg2_ref, h_ref):
    dis = _dis_of(d0_ref, d1_ref)
    agg = jnp.concatenate([acc_ref[0] + hp_ref[0], acc_ref[1] + hp_ref[1]],
                          axis=1)
    x = jnp.maximum(agg * dis + bg1_ref[...], 0.0)
    h2 = jnp.dot(x, wg2_ref[...], preferred_element_type=jnp.float32) * dis
    h_ref[0] = h2[:, :128]
    h_ref[1] = h2[:, 128:]


def _tc3_body(acc_ref, hp_ref, d0_ref, d1_ref, tfeat_ref, bg2_ref, wsp_ref,
              bsp_ref, wa_ref, ba_ref, va_ref, wc1_ref, bc1_ref, wc2_ref,
              bc2_ref, out_ref):
    dis = _dis_of(d0_ref, d1_ref)
    agg = jnp.concatenate([acc_ref[0] + hp_ref[0], acc_ref[1] + hp_ref[1]],
                          axis=1)
    x2 = agg * dis + bg2_ref[...]
    sf = jnp.maximum(
        jnp.dot(x2, wsp_ref[...], preferred_element_type=jnp.float32)
        + bsp_ref[...], 0.0)
    tf = tfeat_ref[...]
    wa = wa_ref[...]
    ba = ba_ref[...]
    va = va_ref[...]
    et = jnp.dot(jnp.tanh(jnp.dot(tf, wa, preferred_element_type=jnp.float32)
                          + ba), va, preferred_element_type=jnp.float32)
    es = jnp.dot(jnp.tanh(jnp.dot(sf, wa, preferred_element_type=jnp.float32)
                          + ba), va, preferred_element_type=jnp.float32)
    a = jax.nn.sigmoid(et - es)
    fused = a * tf + (1.0 - a) * sf
    hc = jnp.maximum(
        jnp.dot(fused, wc1_ref[...], preferred_element_type=jnp.float32)
        + bc1_ref[...], 0.0)
    out_ref[...] = (jnp.dot(hc, wc2_ref[...],
                            preferred_element_type=jnp.float32) + bc2_ref[...])


def _row_spec(rb, cols):
    return pl.BlockSpec((rb, cols), lambda i: (i, 0))


def _plane_spec(rb, cols):
    return pl.BlockSpec((2, rb, cols), lambda i: (0, i, 0))


def _full_spec(shape):
    nd = len(shape)
    return pl.BlockSpec(shape, lambda i, _n=nd: (0,) * _n)


def kernel(temporal_input, spatial_input, edge_index, Wt1, bt1, Wt2, bt2,
           Wg1, bg1, Wg2, bg2, Wsp, bsp, Wa, ba, va, Wc1, bc1, Wc2, bc2):
    nn = spatial_input.shape[0]
    ee = edge_index.shape[1]
    hh = Wt1.shape[1]
    tin = temporal_input.shape[1] * temporal_input.shape[2]
    cc = Wc2.shape[1]
    rb = 1000
    nb = nn // rb
    npad = ((nn + 511) // 512) * 512

    flat = temporal_input.reshape(nn, tin)
    src = edge_index[0]
    dst = edge_index[1]

    degq = _make_deg(npad, ee)(dst)
    if isinstance(degq, (list, tuple)):
        degq = degq[0]
    deg0 = degq[:npad].reshape(npad, 1)
    deg1 = degq[npad:].reshape(npad, 1)

    tc0 = pl.pallas_call(
        _tc0_body,
        grid=(nb,),
        in_specs=[
            _row_spec(rb, tin),
            _full_spec((tin, hh)),
            _full_spec((1, hh)),
            _full_spec((hh, hh)),
            _full_spec((1, hh)),
        ],
        out_specs=[_row_spec(rb, hh)],
        out_shape=[jax.ShapeDtypeStruct((nn, hh), jnp.float32)],
    )
    (tfeat,) = tc0(flat, Wt1, bt1.reshape(1, hh), Wt2, bt2.reshape(1, hh))

    tc1 = pl.pallas_call(
        _tc1_body,
        grid=(nb,),
        in_specs=[
            _row_spec(rb, Wg1.shape[0]),
            _row_spec(rb, 1),
            _row_spec(rb, 1),
            _full_spec((Wg1.shape[0], hh)),
        ],
        out_specs=[_plane_spec(rb, hh // 2)],
        out_shape=[jax.ShapeDtypeStruct((2, npad, hh // 2), jnp.float32)],
    )
    (h1p,) = tc1(spatial_input, deg0, deg1, Wg1)

    conv = _make_conv(npad, ee)

    def _run_conv(hp):
        acc = conv(hp.reshape(2 * npad, hh // 2), src, dst)
        if isinstance(acc, (list, tuple)):
            acc = acc[0]
        return acc.reshape(2, npad, hh // 2)

    acc1 = _run_conv(h1p)

    tc2 = pl.pallas_call(
        _tc2_body,
        grid=(nb,),
        in_specs=[
            _plane_spec(rb, hh // 2),
            _plane_spec(rb, hh // 2),
            _row_spec(rb, 1),
            _row_spec(rb, 1),
            _full_spec((1, hh)),
            _full_spec((hh, hh)),
        ],
        out_specs=[_plane_spec(rb, hh // 2)],
        out_shape=[jax.ShapeDtypeStruct((2, npad, hh // 2), jnp.float32)],
    )
    (h2p,) = tc2(acc1, h1p, deg0, deg1, bg1.reshape(1, hh), Wg2)

    acc2 = _run_conv(h2p)

    tc3 = pl.pallas_call(
        _tc3_body,
        grid=(nb,),
        in_specs=[
            _plane_spec(rb, hh // 2),
            _plane_spec(rb, hh // 2),
            _row_spec(rb, 1),
            _row_spec(rb, 1),
            _row_spec(rb, hh),
            _full_spec((1, hh)),
            _full_spec((hh, hh)),
            _full_spec((1, hh)),
            _full_spec((hh, hh)),
            _full_spec((1, hh)),
            _full_spec((hh, 1)),
            _full_spec((hh, hh // 2)),
            _full_spec((1, hh // 2)),
            _full_spec((hh // 2, cc)),
            _full_spec((1, cc)),
        ],
        out_specs=[_row_spec(rb, cc)],
        out_shape=[jax.ShapeDtypeStruct((nn, cc), jnp.float32)],
    )
    (logits,) = tc3(acc2, h2p, deg0, deg1, tfeat, bg2.reshape(1, hh), Wsp,
                    bsp.reshape(1, hh), Wa, ba.reshape(1, hh),
                    va.reshape(hh, 1), Wc1, bc1.reshape(1, hh // 2), Wc2,
                    bc2.reshape(1, cc))
    return logits
